# baseline (speedup 1.0000x reference)
import functools

import jax
import jax.numpy as jnp
from jax import lax
from jax.experimental import pallas as pl
from jax.experimental.pallas import tpu as pltpu
from jax.experimental.pallas import tpu_sc as plsc

_B = 32
_N_NODES = 50000
_K = 16
_NC = 2
_NS = 16
_NW = _NC * _NS
_GROUP_M = 16
_NGROUPS = _N_NODES // _GROUP_M
_JPW = 100
_IDX_PG = _GROUP_M * _K
_SIGN_MASK = jnp.int32(-(2**31))
_BIG_BITS = jnp.uint32(0x501502F9)


def _sc_body(table_hbm, idx_hbm, out_hbm, tab_sh, idx_r, rows_v, out_v, gsem, osem, isem):
    sid = lax.axis_index("s")
    wid = sid * _NC + lax.axis_index("c")

    def grp(j):
        return jnp.minimum(j * _NW + wid, _NGROUPS - 1)

    def fire_idx(slot4, j):
        g = grp(j)
        for j2 in range(2):
            pltpu.async_copy(
                idx_hbm.at[g * 2 + j2],
                idx_r.at[slot4, j2],
                isem,
            )

    def drain_idx(slot4, j):
        g = grp(j)
        for j2 in range(2):
            pltpu.make_async_copy(
                idx_hbm.at[g * 2 + j2],
                idx_r.at[slot4, j2],
                isem,
            ).wait()

    for b in range(4):
        fire_idx(b, b)

    rows_per_tile = _N_NODES // _NS
    pltpu.sync_copy(
        table_hbm.at[pl.ds(sid * rows_per_tile, rows_per_tile)],
        tab_sh.at[pl.ds(sid * rows_per_tile, rows_per_tile)],
    )
    plsc.subcore_barrier()
    for b in range(4):
        drain_idx(b, b)

    def fire_gathers(slot2, slot4):
        for j2 in range(2):
            pltpu.async_copy(
                tab_sh.at[idx_r.at[slot4, j2]],
                rows_v.at[slot2, pl.ds(j2 * 128, 128)],
                gsem,
            )

    def drain_rows(slot2, slot4):
        for j2 in range(2):
            pltpu.make_async_copy(
                tab_sh.at[idx_r.at[slot4, j2]],
                rows_v.at[slot2, pl.ds(j2 * 128, 128)],
                gsem,
            ).wait()

    def flush_out(oslot, j):
        pltpu.async_copy(
            out_v.at[oslot],
            out_hbm.at[pl.ds(grp(j) * _GROUP_M, _GROUP_M)],
            osem,
        )

    def drain_out(oslot):
        pltpu.make_async_copy(
            out_v.at[oslot],
            out_hbm.at[pl.ds(0, _GROUP_M)],
            osem,
        ).wait()

    def compute(slot, oslot):
        @plsc.parallel_loop(0, _GROUP_M, unroll=2)
        def _mi(mi):
            base = mi * _K
            for h in range(2):
                acci = jnp.zeros((16,), jnp.int32)
                accu = jnp.full((16,), 0xFFFFFFFF, jnp.uint32)
                for n in range(_K):
                    v = rows_v[slot, base + n, pl.ds(h * 16, 16)]
                    vi = lax.bitcast_convert_type(v, jnp.int32)
                    acci = acci ^ vi
                    avu = lax.bitcast_convert_type(vi & 0x7FFFFFFF, jnp.uint32)
                    accu = jnp.minimum(accu, avu - jnp.uint32(1))
                minu = accu + jnp.uint32(1)
                minu = jnp.where(minu == 0, _BIG_BITS, minu)
                ob = lax.bitcast_convert_type(minu, jnp.int32) | (acci & _SIGN_MASK)
                out_v[oslot, mi, pl.ds(h * 16, 16)] = lax.bitcast_convert_type(
                    ob, jnp.float32
                )

    fire_gathers(0, 0)
    fire_gathers(1, 1)

    @pl.loop(0, _JPW, step=4)
    def _outer(jj):
        for b in range(4):
            j = jj + b
            s2 = b % 2
            drain_rows(s2, b)

            @pl.when(j >= 2)
            def _():
                drain_out(s2)

            compute(s2, s2)
            flush_out(s2, j)

            @pl.when(jnp.logical_and(j >= 2, j + 2 < _JPW))
            def _():
                drain_idx((b + 2) % 4, j + 2)

            @pl.when(j + 2 < _JPW)
            def _():
                fire_gathers(s2, (b + 2) % 4)

            @pl.when(j + 4 < _JPW)
            def _():
                fire_idx(b, j + 4)

    for s2 in range(2):
        drain_out(s2)


@functools.cache
def _sc_kernel():
    return pl.kernel(
        _sc_body,
        out_type=jax.ShapeDtypeStruct((_N_NODES, _B), jnp.float32),
        mesh=plsc.VectorSubcoreMesh(core_axis_name="c", subcore_axis_name="s"),
        compiler_params=pltpu.CompilerParams(use_tc_tiling_on_sc=False),
        scratch_types=[
            pltpu.VMEM_SHARED((_N_NODES, _B), jnp.float32),
            pltpu.VMEM((4, 2, 128), jnp.int32),
            pltpu.VMEM((2, _IDX_PG, _B), jnp.float32),
            pltpu.VMEM((2, _GROUP_M, _B), jnp.float32),
            pltpu.SemaphoreType.DMA,
            pltpu.SemaphoreType.DMA,
            pltpu.SemaphoreType.DMA,
        ],
    )


def kernel(input_tensor, check_index_tensor):
    batch, num_nodes = input_tensor.shape
    table = input_tensor.T
    idx = check_index_tensor.astype(jnp.int32).reshape(-1, 128)
    out = _sc_kernel()(table, idx)
    return out.T


# speedup vs baseline: 1.2466x; 1.2466x over previous
import functools

import jax
import jax.numpy as jnp
from jax import lax
from jax.experimental import pallas as pl
from jax.experimental.pallas import tpu as pltpu
from jax.experimental.pallas import tpu_sc as plsc

_B = 32
_N_NODES = 50000
_K = 16
_NC = 2
_NS = 16
_NW = _NC * _NS
_GROUP_M = 16
_NGROUPS = _N_NODES // _GROUP_M
_JPW = 100
_IDX_PG = _GROUP_M * _K
_SIGN_MASK = jnp.int32(-(2**31))
_BIG_BITS = jnp.uint32(0x501502F9)


def _sc_body(table_hbm, idx_hbm, out_hbm, tab_sh, idx_r, rows_v, out_v, gsem, osem, isem):
    sid = lax.axis_index("s")
    wid = sid * _NC + lax.axis_index("c")

    def grp(j):
        return jnp.minimum(j * _NW + wid, _NGROUPS - 1)

    def fire_idx(slot4, j):
        g = grp(j)
        for j2 in range(2):
            pltpu.async_copy(
                idx_hbm.at[g * 2 + j2],
                idx_r.at[slot4, j2],
                isem,
            )

    def drain_idx(slot4, j):
        g = grp(j)
        for j2 in range(2):
            pltpu.make_async_copy(
                idx_hbm.at[g * 2 + j2],
                idx_r.at[slot4, j2],
                isem,
            ).wait()

    for b in range(4):
        fire_idx(b, b)

    rows_per_tile = _N_NODES // _NS
    pltpu.sync_copy(
        table_hbm.at[pl.ds(sid * rows_per_tile, rows_per_tile)],
        tab_sh.at[pl.ds(sid * rows_per_tile, rows_per_tile)],
    )
    plsc.subcore_barrier()
    for b in range(4):
        drain_idx(b, b)

    def fire_gathers(slot2, slot4):
        for j2 in range(2):
            pltpu.async_copy(
                tab_sh.at[idx_r.at[slot4, j2]],
                rows_v.at[slot2, pl.ds(j2 * 128, 128)],
                gsem,
            )

    def drain_rows(slot2, slot4):
        for j2 in range(2):
            pltpu.make_async_copy(
                tab_sh.at[idx_r.at[slot4, j2]],
                rows_v.at[slot2, pl.ds(j2 * 128, 128)],
                gsem,
            ).wait()

    def flush_out(oslot, j):
        pltpu.async_copy(
            out_v.at[oslot],
            out_hbm.at[:, pl.ds(grp(j) * _GROUP_M, _GROUP_M)],
            osem,
        )

    def drain_out(oslot):
        pltpu.make_async_copy(
            out_v.at[oslot],
            out_hbm.at[:, pl.ds(0, _GROUP_M)],
            osem,
        ).wait()

    def compute(slot, oslot):
        lanes = lax.iota(jnp.int32, 16)

        @plsc.parallel_loop(0, _GROUP_M, unroll=2)
        def _mi(mi):
            base = mi * _K
            for h in range(2):
                acci = jnp.zeros((16,), jnp.int32)
                accu = jnp.full((16,), 0xFFFFFFFF, jnp.uint32)
                for n in range(_K):
                    v = rows_v[slot, base + n, pl.ds(h * 16, 16)]
                    vi = lax.bitcast_convert_type(v, jnp.int32)
                    acci = acci ^ vi
                    avu = lax.bitcast_convert_type(vi & 0x7FFFFFFF, jnp.uint32)
                    accu = jnp.minimum(accu, avu - jnp.uint32(1))
                minu = accu + jnp.uint32(1)
                minu = jnp.where(minu == 0, _BIG_BITS, minu)
                ob = lax.bitcast_convert_type(minu, jnp.int32) | (acci & _SIGN_MASK)
                plsc.store_scatter(
                    out_v.at[oslot],
                    [lanes + h * 16, jnp.full((16,), 0, jnp.int32) + mi],
                    lax.bitcast_convert_type(ob, jnp.float32),
                )

    fire_gathers(0, 0)
    fire_gathers(1, 1)

    @pl.loop(0, _JPW, step=4)
    def _outer(jj):
        for b in range(4):
            j = jj + b
            s2 = b % 2
            drain_rows(s2, b)

            @pl.when(j >= 2)
            def _():
                drain_out(s2)

            compute(s2, s2)
            flush_out(s2, j)

            @pl.when(jnp.logical_and(j >= 2, j + 2 < _JPW))
            def _():
                drain_idx((b + 2) % 4, j + 2)

            @pl.when(j + 2 < _JPW)
            def _():
                fire_gathers(s2, (b + 2) % 4)

            @pl.when(j + 4 < _JPW)
            def _():
                fire_idx(b, j + 4)

    for s2 in range(2):
        drain_out(s2)


@functools.cache
def _sc_kernel():
    return pl.kernel(
        _sc_body,
        out_type=jax.ShapeDtypeStruct((_B, _N_NODES), jnp.float32),
        mesh=plsc.VectorSubcoreMesh(core_axis_name="c", subcore_axis_name="s"),
        compiler_params=pltpu.CompilerParams(
            use_tc_tiling_on_sc=False, needs_layout_passes=False
        ),
        scratch_types=[
            pltpu.VMEM_SHARED((_N_NODES, _B), jnp.float32),
            pltpu.VMEM((4, 2, 128), jnp.int32),
            pltpu.VMEM((2, _IDX_PG, _B), jnp.float32),
            pltpu.VMEM((2, _B, _GROUP_M), jnp.float32),
            pltpu.SemaphoreType.DMA,
            pltpu.SemaphoreType.DMA,
            pltpu.SemaphoreType.DMA,
        ],
    )


def kernel(input_tensor, check_index_tensor):
    batch, num_nodes = input_tensor.shape
    table = input_tensor.T
    idx = check_index_tensor.astype(jnp.int32).reshape(-1, 128)
    return _sc_kernel()(table, idx)
